# baseline (speedup 1.0000x reference)
import jax
import jax.numpy as jnp
from jax.experimental import pallas as pl

_N_STEPS = 4
_ROWS_PER_BLOCK = 8

_U32 = jnp.uint32
_ROT_A = (13, 15, 26, 6)
_ROT_B = (17, 29, 16, 24)


def _threefry_bits(ks0, ks1, count):
    ks2 = ks0 ^ ks1 ^ _U32(0x1BD11BDA)
    ks = (ks0, ks1, ks2)
    x0 = jnp.broadcast_to(ks0, count.shape)
    x1 = count + ks1
    for g in range(5):
        rots = _ROT_A if g % 2 == 0 else _ROT_B
        for r in rots:
            x0 = x0 + x1
            x1 = (x1 << _U32(r)) | (x1 >> _U32(32 - r))
            x1 = x1 ^ x0
        x0 = x0 + ks[(g + 1) % 3]
        x1 = x1 + ks[(g + 2) % 3] + _U32(g + 1)
    return x0 ^ x1


def _bits_to_unit_float(bits):
    f = jax.lax.bitcast_convert_type((bits >> _U32(9)) | _U32(0x3F800000),
                                     jnp.float32)
    return f - 1.0


def _lsb_kernel(x_ref, theta_ref, te_ref, keys_ref, out_ref):
    R, D = x_ref.shape
    f32 = jnp.float32
    tiny = jnp.finfo(f32).tiny
    i = pl.program_id(0)

    t = theta_ref[...]
    tmax = jnp.max(t, axis=-1, keepdims=True)
    et = jnp.exp(t - tmax)
    w = et / jnp.sum(et, axis=-1, keepdims=True)
    w0, w1, w2, w3 = (w[:, 0:1], w[:, 1:2], w[:, 2:3], w[:, 3:4])

    def balance(d):
        return (w0 * (d / (1.0 + d)) + w1 * jnp.sqrt(d)
                + w2 * jnp.minimum(1.0, d) + w3 * jnp.maximum(1.0, d))

    te = te_ref[...]
    p_plus = balance(jnp.exp(te))
    p_minus = balance(jnp.exp(-te))
    r_plus = 1.0 / p_plus
    r_minus = 1.0 / p_minus

    x = x_ref[...]
    xb_mask = x > 0.5
    S = jnp.sum(jnp.where(xb_mask, p_minus, p_plus), axis=-1, keepdims=True)

    iota = jax.lax.broadcasted_iota(jnp.int32, (R, D), 1)
    row_iota = jax.lax.broadcasted_iota(jnp.int32, (R, 1), 0)
    ecnt = ((i * R + row_iota) * D + iota).astype(_U32)
    acnt = (i * R + row_iota).astype(_U32)

    for s in range(_N_STEPS):
        kc0 = keys_ref[0, 4 * s + 0]
        kc1 = keys_ref[0, 4 * s + 1]
        ka0 = keys_ref[0, 4 * s + 2]
        ka1 = keys_ref[0, 4 * s + 3]

        m01 = _bits_to_unit_float(_threefry_bits(kc0, kc1, ecnt))
        u = jnp.maximum(tiny, m01 * (1.0 - tiny) + tiny)
        v = -jnp.log(u) * jnp.where(xb_mask, r_minus, r_plus)
        vmin = jnp.min(v, axis=-1, keepdims=True)
        idx = jnp.min(jnp.where(v <= vmin, iota, D), axis=-1, keepdims=True)
        m = (iota == idx).astype(f32)

        xb = jnp.sum(x * m, axis=-1, keepdims=True)
        te_i = jnp.sum(te * m, axis=-1, keepdims=True)
        sgn = 1.0 - 2.0 * xb
        m_term = sgn * te_i
        pf = balance(jnp.exp(m_term))
        pr = balance(jnp.exp(-m_term))
        S_r = S - pf + pr
        la = jnp.minimum(m_term + jnp.log(pr) - jnp.log(S_r)
                         - jnp.log(pf) + jnp.log(S), 0.0)

        ua = _bits_to_unit_float(_threefry_bits(ka0, ka1, acnt))
        acc = jnp.exp(la) > ua

        flip = acc & (m > 0.5)
        x = jnp.where(flip, 1.0 - x, x)
        xb_mask = x > 0.5
        S = jnp.where(acc, S_r, S)

    out_ref[...] = x


def kernel(x, theta, theta_energy):
    B, D = x.shape
    key = jax.random.key(42)
    kws = []
    for i in range(_N_STEPS):
        kc, ka = jax.random.split(jax.random.fold_in(key, i))
        kws.append(jax.random.key_data(kc).astype(_U32))
        kws.append(jax.random.key_data(ka).astype(_U32))
    keys = jnp.concatenate(kws).reshape(1, 4 * _N_STEPS)

    R = _ROWS_PER_BLOCK
    grid = (B // R,)
    out = pl.pallas_call(
        _lsb_kernel,
        grid=grid,
        in_specs=[
            pl.BlockSpec((R, D), lambda i: (i, 0)),
            pl.BlockSpec((1, 4), lambda i: (0, 0)),
            pl.BlockSpec((1, D), lambda i: (0, 0)),
            pl.BlockSpec((1, 4 * _N_STEPS), lambda i: (0, 0)),
        ],
        out_specs=pl.BlockSpec((R, D), lambda i: (i, 0)),
        out_shape=jax.ShapeDtypeStruct((B, D), x.dtype),
    )(x, theta.reshape(1, 4), theta_energy.reshape(1, D), keys)
    return out


# speedup vs baseline: 1.2690x; 1.2690x over previous
import jax
import jax.numpy as jnp
from jax.experimental import pallas as pl

_N_STEPS = 4
_ROWS_PER_BLOCK = 8


def _lsb_kernel(x_ref, ug_ref, ua_ref, theta_ref, te_ref, out_ref):
    R, D = x_ref.shape
    f32 = jnp.float32

    t = theta_ref[...]
    tmax = jnp.max(t, axis=-1, keepdims=True)
    et = jnp.exp(t - tmax)
    w = et / jnp.sum(et, axis=-1, keepdims=True)
    w0, w1, w2, w3 = (w[:, 0:1], w[:, 1:2], w[:, 2:3], w[:, 3:4])

    def balance(d):
        return (w0 * (d / (1.0 + d)) + w1 * jnp.sqrt(d)
                + w2 * jnp.minimum(1.0, d) + w3 * jnp.maximum(1.0, d))

    te = te_ref[...]
    p_plus = balance(jnp.exp(te))
    p_minus = balance(jnp.exp(-te))
    r_plus = 1.0 / p_plus
    r_minus = 1.0 / p_minus

    x = x_ref[...]
    xb_mask = x > 0.5
    S = jnp.sum(jnp.where(xb_mask, p_minus, p_plus), axis=-1, keepdims=True)

    iota = jax.lax.broadcasted_iota(jnp.int32, (R, D), 1)

    for i in range(_N_STEPS):
        u = ug_ref[i]
        v = -jnp.log(u) * jnp.where(xb_mask, r_minus, r_plus)
        vmin = jnp.min(v, axis=-1, keepdims=True)
        idx = jnp.min(jnp.where(v <= vmin, iota, D), axis=-1, keepdims=True)
        m = (iota == idx).astype(f32)

        xb = jnp.sum(x * m, axis=-1, keepdims=True)
        te_i = jnp.sum(te * m, axis=-1, keepdims=True)
        sgn = 1.0 - 2.0 * xb
        m_term = sgn * te_i
        pf = balance(jnp.exp(m_term))
        pr = balance(jnp.exp(-m_term))
        S_r = S - pf + pr
        la = jnp.minimum(m_term + jnp.log(pr) - jnp.log(S_r)
                         - jnp.log(pf) + jnp.log(S), 0.0)
        acc = jnp.exp(la) > ua_ref[:, i:i + 1]

        flip = acc & (m > 0.5)
        x = jnp.where(flip, 1.0 - x, x)
        xb_mask = x > 0.5
        S = jnp.where(acc, S_r, S)

    out_ref[...] = x


def kernel(x, theta, theta_energy):
    B, D = x.shape
    key = jax.random.key(42)
    tiny = jnp.finfo(jnp.float32).tiny
    ugs, uas = [], []
    for i in range(_N_STEPS):
        kc, ka = jax.random.split(jax.random.fold_in(key, i))
        ugs.append(jax.random.uniform(kc, (B, D), jnp.float32,
                                      minval=tiny, maxval=1.0))
        uas.append(jax.random.uniform(ka, (B,), jnp.float32))
    ug = jnp.stack(ugs)
    ua = jnp.stack(uas, axis=1)

    R = _ROWS_PER_BLOCK
    grid = (B // R,)
    out = pl.pallas_call(
        _lsb_kernel,
        grid=grid,
        in_specs=[
            pl.BlockSpec((R, D), lambda i: (i, 0)),
            pl.BlockSpec((_N_STEPS, R, D), lambda i: (0, i, 0)),
            pl.BlockSpec((R, _N_STEPS), lambda i: (i, 0)),
            pl.BlockSpec((1, 4), lambda i: (0, 0)),
            pl.BlockSpec((1, D), lambda i: (0, 0)),
        ],
        out_specs=pl.BlockSpec((R, D), lambda i: (i, 0)),
        out_shape=jax.ShapeDtypeStruct((B, D), x.dtype),
    )(x, ug, ua, theta.reshape(1, 4), theta_energy.reshape(1, D))
    return out
